# baseline (speedup 1.0000x reference)
import functools

import jax
import jax.numpy as jnp
from jax import lax
from jax.experimental import pallas as pl
from jax.experimental.pallas import tpu as pltpu
from jax.experimental.pallas import tpu_sc as plsc

N_NODES = 10000
N_FEAT = 128
N_EDGES = 320000
LANES = 16
N_WORDS = N_FEAT // 2
WORD_CHUNKS = N_WORDS // LANES

_INFO = plsc.get_sparse_core_info()
NC, NS = _INFO.num_cores, _INFO.num_subcores
NW = NC * NS
EDGES_PER_W = N_EDGES // NW
CHUNK = 80
N_CHUNKS = EDGES_PER_W // CHUNK
N_PAIRS = (N_CHUNKS - 1) // 2


def _make_kernel():
    mesh = plsc.VectorSubcoreMesh(core_axis_name="c", subcore_axis_name="s")

    @functools.partial(
        pl.kernel,
        mesh=mesh,
        compiler_params=pltpu.CompilerParams(
            needs_layout_passes=False, use_tc_tiling_on_sc=False),
        out_type=jax.ShapeDtypeStruct((N_EDGES,), jnp.float32),
        scratch_types=[
            pltpu.VMEM((EDGES_PER_W,), jnp.int32),
            pltpu.VMEM((EDGES_PER_W,), jnp.int32),
            pltpu.VMEM((EDGES_PER_W,), jnp.float32),
            pltpu.VMEM((CHUNK, N_WORDS), jnp.int32),
            pltpu.VMEM((CHUNK, N_WORDS), jnp.int32),
            pltpu.VMEM((CHUNK, N_WORDS), jnp.int32),
            pltpu.VMEM((CHUNK, N_WORDS), jnp.int32),
            pltpu.VMEM((CHUNK * LANES,), jnp.float32),
            pltpu.SemaphoreType.DMA,
            pltpu.SemaphoreType.DMA,
            pltpu.SemaphoreType.DMA,
            pltpu.SemaphoreType.DMA,
        ],
    )
    def k(x_hbm, src_hbm, dst_hbm, out_hbm,
          sidx_v, didx_v, outall_v, srows0, drows0, srows1, drows1,
          accbuf_v, ss0, sd0, ss1, sd1):
        wid = lax.axis_index("s") * NC + lax.axis_index("c")
        wbase = wid * EDGES_PER_W
        lanes_iota = lax.iota(jnp.int32, LANES)

        pltpu.sync_copy(src_hbm.at[pl.ds(wbase, EDGES_PER_W)], sidx_v)
        pltpu.sync_copy(dst_hbm.at[pl.ds(wbase, EDGES_PER_W)], didx_v)

        def start(c, srows, drows, sems):
            off = c * CHUNK
            cp1 = pltpu.async_copy(
                x_hbm.at[sidx_v.at[pl.ds(off, CHUNK)]], srows, sems[0])
            cp2 = pltpu.async_copy(
                x_hbm.at[didx_v.at[pl.ds(off, CHUNK)]], drows, sems[1])
            return cp1, cp2

        def wait(srows, drows, sems):
            pltpu.make_async_copy(x_hbm.at[pl.ds(0, CHUNK)], srows,
                                  sems[0]).wait()
            pltpu.make_async_copy(x_hbm.at[pl.ds(0, CHUNK)], drows,
                                  sems[1]).wait()

        def compute(c, srows_v, drows_v):
            def load_edge(eidx):
                return [(plsc.bitcast(srows_v[eidx, pl.ds(j * LANES, LANES)],
                                      jnp.bfloat16),
                         plsc.bitcast(drows_v[eidx, pl.ds(j * LANES, LANES)],
                                      jnp.bfloat16))
                        for j in range(WORD_CHUNKS)]

            def arith(e, regs):
                p = [sj * dj for sj, dj in regs]
                while len(p) > 1:
                    p = [p[i] + p[i + 1] for i in range(0, len(p), 2)]
                u0, u1 = plsc.unpack(
                    p[0], format=plsc.PackFormat.INTERLEAVED)
                accbuf_v[pl.ds(e * LANES, LANES)] = u0 + u1

            def transpose_out(g):
                base = g * LANES * LANES
                cols = [plsc.load_gather(
                            accbuf_v, [lanes_iota * LANES + (base + cc)])
                        for cc in range(LANES)]
                while len(cols) > 1:
                    cols = [cols[i] + cols[i + 1]
                            for i in range(0, len(cols), 2)]
                outall_v[pl.ds(c * CHUNK + g * LANES, LANES)] = cols[0]

            regs = load_edge(0)
            for e in range(1, CHUNK):
                nregs = load_edge(e)
                arith(e - 1, regs)
                regs = nregs
                if e % LANES == 0:
                    transpose_out(e // LANES - 1)
            arith(CHUNK - 1, regs)
            transpose_out(CHUNK // LANES - 1)

        start(0, srows0, drows0, (ss0, sd0))

        def pair_body(j, carry):
            c0 = 2 * j
            start(c0 + 1, srows1, drows1, (ss1, sd1))
            wait(srows0, drows0, (ss0, sd0))
            compute(c0, srows0, drows0)
            start(c0 + 2, srows0, drows0, (ss0, sd0))
            wait(srows1, drows1, (ss1, sd1))
            compute(c0 + 1, srows1, drows1)
            return carry

        lax.fori_loop(0, N_PAIRS, pair_body, 0, unroll=False)
        wait(srows0, drows0, (ss0, sd0))
        compute(N_CHUNKS - 1, srows0, drows0)
        pltpu.sync_copy(outall_v, out_hbm.at[pl.ds(wbase, EDGES_PER_W)])

    return k


_sc_kernel = _make_kernel()


def kernel(x, edge_index):
    ei = edge_index.astype(jnp.int32)
    xb = x.astype(jnp.bfloat16)
    xp = jax.lax.bitcast_convert_type(
        xb.reshape(N_NODES, N_WORDS, 2), jnp.int32)
    positive_edges = _sc_kernel(xp, ei[0], ei[1])
    negative_edges = jnp.array([[0]])
    return (positive_edges, negative_edges)


# speedup vs baseline: 1.2004x; 1.2004x over previous
import functools

import jax
import jax.numpy as jnp
from jax import lax
from jax.experimental import pallas as pl
from jax.experimental.pallas import tpu as pltpu
from jax.experimental.pallas import tpu_sc as plsc

N_NODES = 10000
N_FEAT = 128
N_EDGES = 320000
LANES = 16
BSLICES = N_FEAT // (2 * LANES)

_INFO = plsc.get_sparse_core_info()
NC, NS = _INFO.num_cores, _INFO.num_subcores
NW = NC * NS
EDGES_PER_W = N_EDGES // NW
CHUNK = 80
N_CHUNKS = EDGES_PER_W // CHUNK
N_PAIRS = (N_CHUNKS - 1) // 2
NODE_CHUNKS = N_NODES // CHUNK

_PARAMS = pltpu.CompilerParams(
    needs_layout_passes=False, use_tc_tiling_on_sc=False)
_MESH = plsc.VectorSubcoreMesh(core_axis_name="c", subcore_axis_name="s")


def _row_slices(rows_v, eidx):
    return [rows_v[eidx, pl.ds(j * 2 * LANES, 2 * LANES)]
            for j in range(BSLICES)]


def _sq_acc_row(accbuf_v, e, slices):
    p = []
    for sj in slices:
        u0, u1 = plsc.unpack(sj * sj, format=plsc.PackFormat.INTERLEAVED)
        p += [u0, u1]
    while len(p) > 1:
        p = [p[i] + p[i + 1] for i in range(0, len(p), 2)]
    accbuf_v[pl.ds(e * LANES, LANES)] = p[0]


def _transpose_cols(accbuf_v, lanes_iota):
    cols = [plsc.load_gather(accbuf_v, [lanes_iota * LANES + cc])
            for cc in range(LANES)]
    while len(cols) > 1:
        cols = [cols[i] + cols[i + 1] for i in range(0, len(cols), 2)]
    return cols[0]


def _make_norms_kernel():
    @functools.partial(
        pl.kernel,
        mesh=_MESH,
        compiler_params=_PARAMS,
        out_type=jax.ShapeDtypeStruct((N_NODES,), jnp.float32),
        scratch_types=[
            pltpu.VMEM((CHUNK, N_FEAT), jnp.bfloat16),
            pltpu.VMEM((LANES * LANES,), jnp.float32),
            pltpu.VMEM((CHUNK,), jnp.float32),
        ],
    )
    def k(xb_hbm, out_hbm, rows_v, accbuf_v, outc_v):
        wid = lax.axis_index("s") * NC + lax.axis_index("c")
        lanes_iota = lax.iota(jnp.int32, LANES)
        n_mine = jnp.where(wid < NODE_CHUNKS % NW, NODE_CHUNKS // NW + 1,
                           NODE_CHUNKS // NW)

        def chunk_body(t, carry):
            c = wid + NW * t
            pltpu.sync_copy(xb_hbm.at[pl.ds(c * CHUNK, CHUNK)], rows_v)

            def group_body(g, c2):
                regs = _row_slices(rows_v, g * LANES)
                for e in range(1, LANES):
                    nregs = _row_slices(rows_v, g * LANES + e)
                    _sq_acc_row(accbuf_v, e - 1, regs)
                    regs = nregs
                _sq_acc_row(accbuf_v, LANES - 1, regs)
                outc_v[pl.ds(g * LANES, LANES)] = _transpose_cols(
                    accbuf_v, lanes_iota)
                return c2

            lax.fori_loop(0, CHUNK // LANES, group_body, 0, unroll=False)
            pltpu.sync_copy(outc_v, out_hbm.at[pl.ds(c * CHUNK, CHUNK)])
            return carry

        lax.fori_loop(0, n_mine, chunk_body, 0, unroll=False)

    return k


def _make_main_kernel():
    @functools.partial(
        pl.kernel,
        mesh=_MESH,
        compiler_params=_PARAMS,
        out_type=jax.ShapeDtypeStruct((N_EDGES,), jnp.float32),
        scratch_types=[
            pltpu.VMEM((EDGES_PER_W,), jnp.int32),
            pltpu.VMEM((EDGES_PER_W,), jnp.int32),
            pltpu.VMEM((EDGES_PER_W,), jnp.float32),
            pltpu.VMEM((N_NODES,), jnp.float32),
            pltpu.VMEM((CHUNK, N_FEAT), jnp.bfloat16),
            pltpu.VMEM((CHUNK, N_FEAT), jnp.bfloat16),
            pltpu.VMEM((LANES * LANES,), jnp.float32),
            pltpu.SemaphoreType.DMA,
            pltpu.SemaphoreType.DMA,
            pltpu.SemaphoreType.DMA,
            pltpu.SemaphoreType.DMA,
        ],
    )
    def k(xb_hbm, norms_hbm, src_hbm, dst_hbm, out_hbm,
          sidx_v, didx_v, outall_v, norms_v, rows0, rows1, accbuf_v,
          s10, s20, s11, s21):
        wid = lax.axis_index("s") * NC + lax.axis_index("c")
        wbase = wid * EDGES_PER_W
        lanes_iota = lax.iota(jnp.int32, LANES)

        pltpu.sync_copy(src_hbm.at[pl.ds(wbase, EDGES_PER_W)], sidx_v)
        pltpu.sync_copy(dst_hbm.at[pl.ds(wbase, EDGES_PER_W)], didx_v)
        pltpu.sync_copy(norms_hbm, norms_v)

        def g1(c, rows, sem):
            pltpu.async_copy(
                xb_hbm.at[sidx_v.at[pl.ds(c * CHUNK, CHUNK)]], rows, sem)

        def g2(c, rows, sem):
            pltpu.async_copy(
                xb_hbm.at[didx_v.at[pl.ds(c * CHUNK, CHUNK)]], rows, sem,
                add=True)

        def wait(rows, sem):
            pltpu.make_async_copy(xb_hbm.at[pl.ds(0, CHUNK)], rows,
                                  sem).wait()

        def compute(c, rows_v):
            def group_body(g, c2):
                regs = _row_slices(rows_v, g * LANES)
                for e in range(1, LANES):
                    nregs = _row_slices(rows_v, g * LANES + e)
                    _sq_acc_row(accbuf_v, e - 1, regs)
                    regs = nregs
                _sq_acc_row(accbuf_v, LANES - 1, regs)
                ssq = _transpose_cols(accbuf_v, lanes_iota)
                off = c * CHUNK + g * LANES
                ns = plsc.load_gather(norms_v, [sidx_v[pl.ds(off, LANES)]])
                nd = plsc.load_gather(norms_v, [didx_v[pl.ds(off, LANES)]])
                outall_v[pl.ds(off, LANES)] = 0.5 * ssq - 0.5 * (ns + nd)
                return c2

            lax.fori_loop(0, CHUNK // LANES, group_body, 0, unroll=False)

        g1(0, rows0, s10)
        wait(rows0, s10)
        g2(0, rows0, s20)
        g1(1, rows1, s11)

        def pair_body(j, carry):
            c0 = 2 * j
            wait(rows1, s11)
            g2(c0 + 1, rows1, s21)
            wait(rows0, s20)
            compute(c0, rows0)
            g1(c0 + 2, rows0, s10)
            wait(rows1, s21)
            compute(c0 + 1, rows1)

            @pl.when(j < N_PAIRS - 1)
            def _():
                g1(c0 + 3, rows1, s11)

            wait(rows0, s10)
            g2(c0 + 2, rows0, s20)
            return carry

        lax.fori_loop(0, N_PAIRS, pair_body, 0, unroll=False)
        wait(rows0, s20)
        compute(N_CHUNKS - 1, rows0)
        pltpu.sync_copy(outall_v, out_hbm.at[pl.ds(wbase, EDGES_PER_W)])

    return k


_norms_kernel = _make_norms_kernel()
_main_kernel = _make_main_kernel()


def kernel(x, edge_index):
    ei = edge_index.astype(jnp.int32)
    xb = x.astype(jnp.bfloat16)
    norms = _norms_kernel(xb)
    positive_edges = _main_kernel(xb, norms, ei[0], ei[1])
    negative_edges = jnp.array([[0]])
    return (positive_edges, negative_edges)


# speedup vs baseline: 1.4136x; 1.1775x over previous
import functools

import jax
import jax.numpy as jnp
from jax import lax
from jax.experimental import pallas as pl
from jax.experimental.pallas import tpu as pltpu
from jax.experimental.pallas import tpu_sc as plsc

N_NODES = 10000
N_FEAT = 128
N_EDGES = 320000
LANES = 16
N_WORDS = N_FEAT // 2
WORD_CHUNKS = N_WORDS // LANES

_INFO = plsc.get_sparse_core_info()
NC, NS = _INFO.num_cores, _INFO.num_subcores
NW = NC * NS
EDGES_PER_W = N_EDGES // NW
CHUNK = 80
N_CHUNKS = EDGES_PER_W // CHUNK
N_PAIRS = (N_CHUNKS - 1) // 2
LAG = 2


def _make_kernel():
    mesh = plsc.VectorSubcoreMesh(core_axis_name="c", subcore_axis_name="s")

    @functools.partial(
        pl.kernel,
        mesh=mesh,
        compiler_params=pltpu.CompilerParams(
            needs_layout_passes=False, use_tc_tiling_on_sc=False),
        out_type=jax.ShapeDtypeStruct((N_EDGES,), jnp.float32),
        scratch_types=[
            pltpu.VMEM((EDGES_PER_W,), jnp.int32),
            pltpu.VMEM((EDGES_PER_W,), jnp.int32),
            pltpu.VMEM((EDGES_PER_W,), jnp.float32),
            pltpu.VMEM((CHUNK, N_WORDS), jnp.int32),
            pltpu.VMEM((CHUNK, N_WORDS), jnp.int32),
            pltpu.VMEM((CHUNK, N_WORDS), jnp.int32),
            pltpu.VMEM((CHUNK, N_WORDS), jnp.int32),
            pltpu.VMEM((CHUNK * LANES,), jnp.float32),
            pltpu.SemaphoreType.DMA,
            pltpu.SemaphoreType.DMA,
            pltpu.SemaphoreType.DMA,
            pltpu.SemaphoreType.DMA,
        ],
    )
    def k(x_hbm, src_hbm, dst_hbm, out_hbm,
          sidx_v, didx_v, outall_v, srows0, drows0, srows1, drows1,
          accbuf_v, ss0, sd0, ss1, sd1):
        wid = lax.axis_index("s") * NC + lax.axis_index("c")
        wbase = wid * EDGES_PER_W
        lanes_iota = lax.iota(jnp.int32, LANES)

        pltpu.sync_copy(src_hbm.at[pl.ds(wbase, EDGES_PER_W)], sidx_v)
        pltpu.sync_copy(dst_hbm.at[pl.ds(wbase, EDGES_PER_W)], didx_v)

        def start(c, srows, drows, sems):
            off = c * CHUNK
            cp1 = pltpu.async_copy(
                x_hbm.at[sidx_v.at[pl.ds(off, CHUNK)]], srows, sems[0])
            cp2 = pltpu.async_copy(
                x_hbm.at[didx_v.at[pl.ds(off, CHUNK)]], drows, sems[1])
            return cp1, cp2

        def wait(srows, drows, sems):
            pltpu.make_async_copy(x_hbm.at[pl.ds(0, CHUNK)], srows,
                                  sems[0]).wait()
            pltpu.make_async_copy(x_hbm.at[pl.ds(0, CHUNK)], drows,
                                  sems[1]).wait()

        def compute(c, srows_v, drows_v):
            def load_edge(eidx):
                return [(plsc.bitcast(srows_v[eidx, pl.ds(j * LANES, LANES)],
                                      jnp.bfloat16),
                         plsc.bitcast(drows_v[eidx, pl.ds(j * LANES, LANES)],
                                      jnp.bfloat16))
                        for j in range(WORD_CHUNKS)]

            def arith(row, regs):
                p = [sj * dj for sj, dj in regs]
                while len(p) > 1:
                    p = [p[i] + p[i + 1] for i in range(0, len(p), 2)]
                u0, u1 = plsc.unpack(
                    p[0], format=plsc.PackFormat.INTERLEAVED)
                accbuf_v[pl.ds(row * LANES, LANES)] = u0 + u1

            def group_body(g, c2):
                pipe = [load_edge(g * LANES + e) for e in range(LAG)]
                for e in range(LAG, LANES):
                    pipe.append(load_edge(g * LANES + e))
                    arith(e - LAG, pipe.pop(0))
                for e in range(LANES - LAG, LANES):
                    arith(e, pipe.pop(0))
                base = g * LANES * LANES
                cols = [plsc.load_gather(
                            accbuf_v, [lanes_iota * LANES + (base + cc)])
                        for cc in range(LANES)]
                while len(cols) > 1:
                    cols = [cols[i] + cols[i + 1]
                            for i in range(0, len(cols), 2)]
                outall_v[pl.ds(c * CHUNK + g * LANES, LANES)] = cols[0]
                return c2

            lax.fori_loop(0, CHUNK // LANES, group_body, 0, unroll=False)

        start(0, srows0, drows0, (ss0, sd0))

        def pair_body(j, carry):
            c0 = 2 * j
            start(c0 + 1, srows1, drows1, (ss1, sd1))
            wait(srows0, drows0, (ss0, sd0))
            compute(c0, srows0, drows0)
            start(c0 + 2, srows0, drows0, (ss0, sd0))
            wait(srows1, drows1, (ss1, sd1))
            compute(c0 + 1, srows1, drows1)
            return carry

        lax.fori_loop(0, N_PAIRS, pair_body, 0, unroll=False)
        wait(srows0, drows0, (ss0, sd0))
        compute(N_CHUNKS - 1, srows0, drows0)
        pltpu.sync_copy(outall_v, out_hbm.at[pl.ds(wbase, EDGES_PER_W)])

    return k


_sc_kernel = _make_kernel()


def kernel(x, edge_index):
    ei = edge_index.astype(jnp.int32)
    xb = x.astype(jnp.bfloat16)
    xp = jax.lax.bitcast_convert_type(
        xb.reshape(N_NODES, N_WORDS, 2), jnp.int32)
    positive_edges = _sc_kernel(xp, ei[0], ei[1])
    negative_edges = jnp.array([[0]])
    return (positive_edges, negative_edges)
